# Initial kernel scaffold; baseline (speedup 1.0000x reference)
import jax
import jax.numpy as jnp
from jax.experimental import pallas as pl


def kernel(content_all, content_all_mask, additional_bs, additional_bs_mask, content_prev_sep, pos_tags, wte, pos_table, token_weights, W_enc):
    raise NotImplementedError("write your pallas kernel here")


# speedup vs baseline: 1.4600x; 1.4600x over previous
import functools

import jax
import jax.numpy as jnp
import numpy as np
from jax import lax
from jax.experimental import pallas as pl
from jax.experimental.pallas import tpu as pltpu
from jax.experimental.pallas import tpu_sc as plsc

B = 16
L = 200
LA = 10
D_IN = 1024
D = 128
P = 75
S_OUT = 1 + LA + 2 + L + P

_HALF = L // 2
_WCHUNK = 104
_PCHUNK = 80
_ACHUNK = 16


def _dst_tables():
    base = np.arange(B, dtype=np.int32)[:, None] * S_OUT
    h0 = np.concatenate([[0], 13 + np.arange(_HALF), [112] * (_WCHUNK - _HALF - 1)]).astype(np.int32)
    h1 = np.concatenate([113 + np.arange(_HALF), [212] * (_WCHUNK - _HALF)]).astype(np.int32)
    dstw = np.stack([h0[None, :] + base, h1[None, :] + base], axis=1).reshape(-1)
    pos = np.concatenate([213 + np.arange(P), [287] * (_PCHUNK - P)]).astype(np.int32)
    dstp = (pos[None, :] + base).reshape(-1)
    arow = np.concatenate([1 + np.arange(12), [12] * 4]).astype(np.int32)
    dsta = (arow[None, :] + base).reshape(-1)
    return dstw, dstp, dsta


_DSTW, _DSTP, _DSTA = _dst_tables()


def _tc_body(abs_ref, w_ref, tw_ref, absw_ref, mask_ref):
    a = abs_ref[...].reshape(B * LA, D_IN)
    proj = lax.dot_general(a, w_ref[...], (((1,), (0,)), ((), ())),
                           preferred_element_type=jnp.float32)
    tw = tw_ref[...]
    t0 = jnp.broadcast_to(tw[0][None, None, :], (B, 1, D))
    t1 = jnp.broadcast_to(tw[1][None, None, :], (B, 5, D))
    absw_ref[...] = jnp.concatenate([proj.reshape(B, LA, D), t0, t1], axis=1)
    mask_ref[...] = jnp.ones((B, S_OUT), jnp.float32)


_tc_call = pl.pallas_call(
    _tc_body,
    out_shape=(
        jax.ShapeDtypeStruct((B, _ACHUNK, D), jnp.float32),
        jax.ShapeDtypeStruct((B, S_OUT), jnp.float32),
    ),
)


def _sc_body(wte, pos_table, absw, idx_wte, dstw, pos_idx, dstp, dsta, out,
             idx_v, dstw_v, rows_v, pidx_v, dstp_v, prows_v, dsta_v, arows_v, sem):
    c = lax.axis_index("c")
    s = lax.axis_index("s")
    b = s
    wid = s * 2 + c

    pltpu.sync_copy(idx_wte.at[pl.ds(wid * _WCHUNK, _WCHUNK)], idx_v)
    pltpu.sync_copy(dstw.at[pl.ds(wid * _WCHUNK, _WCHUNK)], dstw_v)
    pltpu.async_copy(wte.at[idx_v], rows_v, sem).wait()
    pltpu.async_copy(rows_v, out.at[dstw_v], sem).wait()

    @pl.when(c == 1)
    def _():
        pltpu.sync_copy(pos_idx.at[pl.ds(b * _PCHUNK, _PCHUNK)], pidx_v)
        pltpu.sync_copy(dstp.at[pl.ds(b * _PCHUNK, _PCHUNK)], dstp_v)
        pltpu.async_copy(pos_table.at[pidx_v], prows_v, sem).wait()
        pltpu.async_copy(prows_v, out.at[dstp_v], sem).wait()
        pltpu.sync_copy(absw.at[b], arows_v)
        pltpu.sync_copy(dsta.at[pl.ds(b * _ACHUNK, _ACHUNK)], dsta_v)
        pltpu.async_copy(arows_v, out.at[dsta_v], sem).wait()


_sc_call = functools.partial(
    pl.kernel,
    out_type=jax.ShapeDtypeStruct((B * S_OUT, D), jnp.float32),
    mesh=plsc.VectorSubcoreMesh(core_axis_name="c", subcore_axis_name="s"),
    scratch_types=[
        pltpu.VMEM((_WCHUNK,), jnp.int32),
        pltpu.VMEM((_WCHUNK,), jnp.int32),
        pltpu.VMEM((_WCHUNK, D), jnp.float32),
        pltpu.VMEM((_PCHUNK,), jnp.int32),
        pltpu.VMEM((_PCHUNK,), jnp.int32),
        pltpu.VMEM((_PCHUNK, D), jnp.float32),
        pltpu.VMEM((_ACHUNK,), jnp.int32),
        pltpu.VMEM((_ACHUNK, D), jnp.float32),
        pltpu.SemaphoreType.DMA,
    ],
)(_sc_body)


def kernel(content_all, content_all_mask, additional_bs, additional_bs_mask,
           content_prev_sep, pos_tags, wte, pos_table, token_weights, W_enc):
    h0 = jnp.concatenate(
        [content_prev_sep[:, :1], content_all[:, :_HALF],
         jnp.tile(content_all[:, _HALF - 1:_HALF], (1, _WCHUNK - _HALF - 1))], axis=1)
    h1 = jnp.concatenate(
        [content_all[:, _HALF:],
         jnp.tile(content_all[:, L - 1:L], (1, _WCHUNK - _HALF))], axis=1)
    idx_wte = jnp.stack([h0, h1], axis=1).reshape(-1)
    pos_idx = jnp.concatenate(
        [pos_tags, jnp.tile(pos_tags[:, P - 1:P], (1, _PCHUNK - P))], axis=1).reshape(-1)

    absw, mask = _tc_call(additional_bs, W_enc, token_weights)
    content = _sc_call(wte, pos_table, absw, idx_wte, jnp.asarray(_DSTW),
                       pos_idx, jnp.asarray(_DSTP), jnp.asarray(_DSTA))
    return content.reshape(B, S_OUT, D), mask


# speedup vs baseline: 1.8420x; 1.2617x over previous
import functools

import jax
import jax.numpy as jnp
import numpy as np
from jax import lax
from jax.experimental import pallas as pl
from jax.experimental.pallas import tpu as pltpu
from jax.experimental.pallas import tpu_sc as plsc

B = 16
L = 200
LA = 10
D_IN = 1024
D = 128
P = 75
S_OUT = 1 + LA + 2 + L + P

_HALF = L // 2
_WLOAD = 104
_PSPLIT = 40
_ACHUNK = 16


def _dst_tables():
    base = np.arange(B, dtype=np.int32)[:, None] * S_OUT
    h0 = np.concatenate([[0], 13 + np.arange(_HALF), [112] * (_WLOAD - _HALF - 1)]).astype(np.int32)
    h1 = np.concatenate([113 + np.arange(_HALF), [212] * (_WLOAD - _HALF)]).astype(np.int32)
    dstw = np.concatenate([(h0[None, :] + base).reshape(-1),
                           (h1[None, :] + base).reshape(-1)])
    pos = np.concatenate([213 + np.arange(P), [287] * (2 * _PSPLIT - P)]).astype(np.int32)
    dstp = (pos[None, :] + base).reshape(-1)
    arow = np.concatenate([1 + np.arange(12), [12] * (_ACHUNK - 12)]).astype(np.int32)
    dsta = (arow[None, :] + base).reshape(-1)
    return dstw, dstp, dsta


_DSTW, _DSTP, _DSTA = _dst_tables()


def _tc_body(ca_ref, sep_ref, pt_ref, abs_ref, w_ref, tw_ref,
             absw_ref, mask_ref, idxw_ref, idxp_ref):
    a = abs_ref[...].reshape(B * LA, D_IN)
    proj = lax.dot_general(a, w_ref[...], (((1,), (0,)), ((), ())),
                           preferred_element_type=jnp.float32)
    tw = tw_ref[...]
    t0 = jnp.broadcast_to(tw[0][None, None, :], (B, 1, D))
    t1 = jnp.broadcast_to(tw[1][None, None, :], (B, 5, D))
    absw_ref[...] = jnp.concatenate([proj.reshape(B, LA, D), t0, t1], axis=1)
    mask_ref[...] = jnp.ones((B, S_OUT), jnp.float32)
    ca = ca_ref[...]
    h0 = jnp.concatenate(
        [sep_ref[...][:, :1], ca[:, :_HALF],
         jnp.broadcast_to(ca[:, _HALF - 1:_HALF], (B, 128 - _HALF - 1))], axis=1)
    h1 = jnp.concatenate(
        [ca[:, _HALF:], jnp.broadcast_to(ca[:, L - 1:L], (B, 128 - _HALF))], axis=1)
    idxw_ref[...] = jnp.concatenate([h0, h1], axis=0)
    pt = pt_ref[...]
    idxp_ref[...] = jnp.concatenate(
        [pt, jnp.broadcast_to(pt[:, P - 1:P], (B, 128 - P))], axis=1)


_tc_call = pl.pallas_call(
    _tc_body,
    out_shape=(
        jax.ShapeDtypeStruct((B, _ACHUNK, D), jnp.float32),
        jax.ShapeDtypeStruct((B, S_OUT), jnp.float32),
        jax.ShapeDtypeStruct((2 * B, 128), jnp.int32),
        jax.ShapeDtypeStruct((B, 128), jnp.int32),
    ),
)


def _sc_body(wte, pos_table, absw, idx_wte, dstw, pos_idx, dstp, dsta, out,
             idx_v, dstw_v, rows_v, pidx_v, dstp_v, prows_v, dsta_v, arows_v,
             sem0, sem1, sem2):
    c = lax.axis_index("c")
    s = lax.axis_index("s")
    b = s
    chunk = c * B + s

    l1 = pltpu.async_copy(idx_wte.at[pl.ds(chunk * 128, _WLOAD)], idx_v, sem0)
    l2 = pltpu.async_copy(pos_idx.at[pl.ds(b * 128 + c * _PSPLIT, _PSPLIT)], pidx_v, sem0)
    l3 = pltpu.async_copy(dstw.at[pl.ds(chunk * _WLOAD, _WLOAD)], dstw_v, sem1)
    l4 = pltpu.async_copy(dstp.at[pl.ds(b * 2 * _PSPLIT + c * _PSPLIT, _PSPLIT)], dstp_v, sem1)
    l1.wait()
    l2.wait()
    g1 = pltpu.async_copy(wte.at[idx_v], rows_v, sem0)
    g2 = pltpu.async_copy(pos_table.at[pidx_v], prows_v, sem0)
    l3.wait()
    l4.wait()
    g1.wait()
    g2.wait()
    s1 = pltpu.async_copy(rows_v, out.at[dstw_v], sem2)
    s2 = pltpu.async_copy(prows_v, out.at[dstp_v], sem2)

    @pl.when(c == 1)
    def _():
        a1 = pltpu.async_copy(absw.at[b], arows_v, sem0)
        a2 = pltpu.async_copy(dsta.at[pl.ds(b * _ACHUNK, _ACHUNK)], dsta_v, sem1)
        a1.wait()
        a2.wait()
        pltpu.async_copy(arows_v, out.at[dsta_v], sem2).wait()

    s1.wait()
    s2.wait()


_sc_call = functools.partial(
    pl.kernel,
    out_type=jax.ShapeDtypeStruct((B * S_OUT, D), jnp.float32),
    mesh=plsc.VectorSubcoreMesh(core_axis_name="c", subcore_axis_name="s"),
    scratch_types=[
        pltpu.VMEM((_WLOAD,), jnp.int32),
        pltpu.VMEM((_WLOAD,), jnp.int32),
        pltpu.VMEM((_WLOAD, D), jnp.float32),
        pltpu.VMEM((_PSPLIT,), jnp.int32),
        pltpu.VMEM((_PSPLIT,), jnp.int32),
        pltpu.VMEM((_PSPLIT, D), jnp.float32),
        pltpu.VMEM((_ACHUNK,), jnp.int32),
        pltpu.VMEM((_ACHUNK, D), jnp.float32),
        pltpu.SemaphoreType.DMA,
        pltpu.SemaphoreType.DMA,
        pltpu.SemaphoreType.DMA,
    ],
)(_sc_body)


def kernel(content_all, content_all_mask, additional_bs, additional_bs_mask,
           content_prev_sep, pos_tags, wte, pos_table, token_weights, W_enc):
    absw, mask, idxw, idxp = _tc_call(content_all, content_prev_sep, pos_tags,
                                      additional_bs, W_enc, token_weights)
    content = _sc_call(wte, pos_table, absw,
                       idxw.reshape(-1), jnp.asarray(_DSTW),
                       idxp.reshape(-1), jnp.asarray(_DSTP), jnp.asarray(_DSTA))
    return content.reshape(B, S_OUT, D), mask


# speedup vs baseline: 1.9350x; 1.0505x over previous
import functools

import jax
import jax.numpy as jnp
import numpy as np
from jax import lax
from jax.experimental import pallas as pl
from jax.experimental.pallas import tpu as pltpu
from jax.experimental.pallas import tpu_sc as plsc

B = 16
L = 200
LA = 10
D_IN = 1024
D = 128
P = 75
S_OUT = 1 + LA + 2 + L + P

_HALF = L // 2
_WLOAD = 104
_PSPLIT = 40
_ASPLIT = 8
_NROWS = _WLOAD + _PSPLIT + _ASPLIT
_ACHUNK = 16


def _dst_tables():
    base = np.arange(B, dtype=np.int32)[:, None] * S_OUT
    h0 = np.concatenate([[0], 13 + np.arange(_HALF), [112] * (_WLOAD - _HALF - 1)])
    h1 = np.concatenate([113 + np.arange(_HALF), [212] * (_WLOAD - _HALF)])
    pos = np.concatenate([213 + np.arange(P), [287] * (2 * _PSPLIT - P)])
    arow = np.concatenate([1 + np.arange(12), [12] * (2 * _ASPLIT - 12)])
    c0 = np.concatenate([h0, pos[:_PSPLIT], arow[:_ASPLIT]]).astype(np.int32)
    c1 = np.concatenate([h1, pos[_PSPLIT:], arow[_ASPLIT:]]).astype(np.int32)
    return np.concatenate([(c0[None, :] + base).reshape(-1),
                           (c1[None, :] + base).reshape(-1)])


_DST = _dst_tables()


def _tc_body(ca_ref, sep_ref, pt_ref, abs_ref, w_ref, tw_ref,
             absw_ref, mask_ref, idxw_ref, idxp_ref):
    a = abs_ref[...].reshape(B * LA, D_IN)
    proj = lax.dot_general(a, w_ref[...], (((1,), (0,)), ((), ())),
                           preferred_element_type=jnp.float32)
    tw = tw_ref[...]
    t0 = jnp.broadcast_to(tw[0][None, None, :], (B, 1, D))
    t1 = jnp.broadcast_to(tw[1][None, None, :], (B, 5, D))
    absw_ref[...] = jnp.concatenate([proj.reshape(B, LA, D), t0, t1], axis=1)
    mask_ref[...] = jnp.ones((B, S_OUT), jnp.float32)
    ca = ca_ref[...]
    h0 = jnp.concatenate(
        [sep_ref[...][:, :1], ca[:, :_HALF],
         jnp.broadcast_to(ca[:, _HALF - 1:_HALF], (B, 128 - _HALF - 1))], axis=1)
    h1 = jnp.concatenate(
        [ca[:, _HALF:], jnp.broadcast_to(ca[:, L - 1:L], (B, 128 - _HALF))], axis=1)
    idxw_ref[...] = jnp.concatenate([h0, h1], axis=0)
    pt = pt_ref[...]
    idxp_ref[...] = jnp.concatenate(
        [pt, jnp.broadcast_to(pt[:, P - 1:P], (B, 128 - P))], axis=1)


_tc_call = pl.pallas_call(
    _tc_body,
    out_shape=(
        jax.ShapeDtypeStruct((B, _ACHUNK, D), jnp.float32),
        jax.ShapeDtypeStruct((B, S_OUT), jnp.float32),
        jax.ShapeDtypeStruct((2 * B, 128), jnp.int32),
        jax.ShapeDtypeStruct((B, 128), jnp.int32),
    ),
)


def _sc_body(wte, pos_table, absw, idx_wte, pos_idx, dst, out,
             idx_v, pidx_v, dst_v, rows_v, sem0, sem1, sem2):
    c = lax.axis_index("c")
    s = lax.axis_index("s")
    b = s
    chunk = c * B + s

    a1 = pltpu.async_copy(absw.at[b, pl.ds(c * _ASPLIT, _ASPLIT)],
                          rows_v.at[pl.ds(_WLOAD + _PSPLIT, _ASPLIT)], sem2)
    l1 = pltpu.async_copy(idx_wte.at[pl.ds(chunk * 128, _WLOAD)], idx_v, sem0)
    l2 = pltpu.async_copy(pos_idx.at[pl.ds(b * 128 + c * _PSPLIT, _PSPLIT)], pidx_v, sem0)
    l3 = pltpu.async_copy(dst.at[pl.ds(chunk * _NROWS, _NROWS)], dst_v, sem1)
    l1.wait()
    l2.wait()
    g1 = pltpu.async_copy(wte.at[idx_v], rows_v.at[pl.ds(0, _WLOAD)], sem0)
    g2 = pltpu.async_copy(pos_table.at[pidx_v], rows_v.at[pl.ds(_WLOAD, _PSPLIT)], sem0)
    l3.wait()
    a1.wait()
    g1.wait()
    g2.wait()
    pltpu.async_copy(rows_v, out.at[dst_v], sem2).wait()


_sc_call = functools.partial(
    pl.kernel,
    out_type=jax.ShapeDtypeStruct((B * S_OUT, D), jnp.float32),
    mesh=plsc.VectorSubcoreMesh(core_axis_name="c", subcore_axis_name="s"),
    scratch_types=[
        pltpu.VMEM((_WLOAD,), jnp.int32),
        pltpu.VMEM((_PSPLIT,), jnp.int32),
        pltpu.VMEM((_NROWS,), jnp.int32),
        pltpu.VMEM((_NROWS, D), jnp.float32),
        pltpu.SemaphoreType.DMA,
        pltpu.SemaphoreType.DMA,
        pltpu.SemaphoreType.DMA,
    ],
)(_sc_body)


def kernel(content_all, content_all_mask, additional_bs, additional_bs_mask,
           content_prev_sep, pos_tags, wte, pos_table, token_weights, W_enc):
    absw, mask, idxw, idxp = _tc_call(content_all, content_prev_sep, pos_tags,
                                      additional_bs, W_enc, token_weights)
    content = _sc_call(wte, pos_table, absw,
                       idxw.reshape(-1), idxp.reshape(-1), jnp.asarray(_DST))
    return content.reshape(B, S_OUT, D), mask
